# baseline (speedup 1.0000x reference)
import functools

import jax
import jax.numpy as jnp
from jax import lax
from jax.experimental import pallas as pl
from jax.experimental.pallas import tpu as pltpu
from jax.experimental.pallas import tpu_sc as plsc


def _embed_kernel(n_rows, seq_len, d_model, n_workers, n_cores):
    rows_per_w = n_rows // n_workers
    chunk = 32
    n_chunks = rows_per_w // chunk
    lanes = 16
    vregs_per_row = d_model // lanes

    mesh = plsc.VectorSubcoreMesh(core_axis_name="c", subcore_axis_name="s")

    @functools.partial(
        pl.kernel,
        mesh=mesh,
        out_type=jax.ShapeDtypeStruct((n_rows, d_model), jnp.float32),
        scratch_types=[
            pltpu.VMEM((rows_per_w,), jnp.int32),
            pltpu.VMEM((chunk, d_model), jnp.float32),
            pltpu.VMEM((chunk, d_model), jnp.float32),
            pltpu.SemaphoreType.DMA,
        ],
    )
    def k(ids_hbm, tok_hbm, pos_hbm, out_hbm, idx_v, tbuf, pbuf, sem):
        wid = lax.axis_index("s") * n_cores + lax.axis_index("c")
        base = wid * rows_per_w
        p0 = lax.rem(base, seq_len)
        pltpu.sync_copy(ids_hbm.at[pl.ds(base, rows_per_w)], idx_v)

        def chunk_body(g, carry):
            row0 = g * chunk
            gather = pltpu.async_copy(
                tok_hbm.at[idx_v.at[pl.ds(row0, chunk)]], tbuf, sem
            )
            pltpu.sync_copy(pos_hbm.at[pl.ds(p0 + row0, chunk)], pbuf)
            gather.wait()

            def row_body(r, c):
                for j in range(vregs_per_row):
                    sl = pl.ds(j * lanes, lanes)
                    tbuf[r, sl] = tbuf[r, sl] + pbuf[r, sl]
                return c

            lax.fori_loop(0, chunk, row_body, 0)
            pltpu.sync_copy(tbuf, out_hbm.at[pl.ds(base + row0, chunk)])
            return carry

        lax.fori_loop(0, n_chunks, chunk_body, 0)

    return k


def kernel(x_ids, token_table, pos_table):
    b, t = x_ids.shape
    _, d = token_table.shape
    flat_ids = x_ids.reshape(b * t).astype(jnp.int32)
    info = plsc.get_sparse_core_info()
    n_workers = info.num_cores * info.num_subcores
    k = _embed_kernel(b * t, t, d, n_workers, info.num_cores)
    out = k(flat_ids, token_table, pos_table)
    return out.reshape(b, t, d)


# speedup vs baseline: 1.1096x; 1.1096x over previous
import functools

import jax
import jax.numpy as jnp
from jax import lax
from jax.experimental import pallas as pl
from jax.experimental.pallas import tpu as pltpu
from jax.experimental.pallas import tpu_sc as plsc

_LANES = 16
_CHUNK = 16


def _embed_kernel(n_batch, seq_len, d_model, n_workers, n_cores):
    pos_per_w = seq_len // n_workers
    n_chunks = pos_per_w // _CHUNK
    vregs_per_row = d_model // _LANES
    assert n_chunks % 2 == 0 and n_batch % 2 == 0

    mesh = plsc.VectorSubcoreMesh(core_axis_name="c", subcore_axis_name="s")

    @functools.partial(
        pl.kernel,
        mesh=mesh,
        out_type=jax.ShapeDtypeStruct((n_batch * seq_len, d_model), jnp.float32),
        scratch_types=[
            pltpu.VMEM((n_batch, pos_per_w), jnp.int32),
            pltpu.VMEM((2, _CHUNK, d_model), jnp.float32),
            pltpu.VMEM((2, _CHUNK, d_model), jnp.float32),
            pltpu.SemaphoreType.DMA,
            pltpu.SemaphoreType.DMA,
            pltpu.SemaphoreType.DMA,
            pltpu.SemaphoreType.DMA,
            pltpu.SemaphoreType.DMA,
            pltpu.SemaphoreType.DMA,
        ],
    )
    def k(ids_hbm, tok_hbm, pos_hbm, out_hbm, idx_v, tbuf, pbuf,
          g0, g1, o0, o1, p0, p1):
        wid = lax.axis_index("s") * n_cores + lax.axis_index("c")
        pbase = wid * pos_per_w
        gsem, osem, psem = (g0, g1), (o0, o1), (p0, p1)

        for b in range(n_batch):
            pltpu.sync_copy(
                ids_hbm.at[pl.ds(b * seq_len + pbase, pos_per_w)], idx_v.at[b]
            )

        def fire_pos(g, pg):
            pltpu.async_copy(
                pos_hbm.at[pl.ds(pbase + g * _CHUNK, _CHUNK)], pbuf.at[pg],
                psem[pg],
            )

        def wait_pos(pg):
            pltpu.make_async_copy(
                pos_hbm.at[pl.ds(0, _CHUNK)], pbuf.at[pg], psem[pg]
            ).wait()

        def fire_gather(g, b, par):
            pltpu.async_copy(
                tok_hbm.at[idx_v.at[b, pl.ds(g * _CHUNK, _CHUNK)]],
                tbuf.at[par], gsem[par],
            )

        def wait_gather(par):
            pltpu.make_async_copy(
                tok_hbm.at[pl.ds(0, _CHUNK)], tbuf.at[par], gsem[par]
            ).wait()

        def fire_out(g, b, par):
            pltpu.async_copy(
                tbuf.at[par],
                out_hbm.at[pl.ds(b * seq_len + pbase + g * _CHUNK, _CHUNK)],
                osem[par],
            )

        def wait_out(par):
            pltpu.make_async_copy(
                tbuf.at[par], out_hbm.at[pl.ds(0, _CHUNK)], osem[par]
            ).wait()

        def add_pos(par, pg):
            def row_body(r, c):
                for j in range(vregs_per_row):
                    sl = pl.ds(j * _LANES, _LANES)
                    tbuf[par, r, sl] = tbuf[par, r, sl] + pbuf[pg, r, sl]
                return c

            lax.fori_loop(0, _CHUNK, row_body, 0)

        fire_pos(0, 0)
        fire_gather(0, 0, 0)

        def gg_body(gg, carry):
            for g_par in (0, 1):
                g = 2 * gg + g_par
                pg = g_par
                for b in range(n_batch):
                    par = b % 2
                    wait_gather(par)
                    if b == 0:
                        wait_pos(pg)
                        if g_par == 1:
                            @pl.when(gg < n_chunks // 2 - 1)
                            def _():
                                fire_pos(g + 1, 1 - pg)
                        else:
                            fire_pos(g + 1, 1 - pg)
                    if b == 0 and g_par == 0:
                        @pl.when(gg > 0)
                        def _():
                            wait_out(1 - par)
                    else:
                        wait_out(1 - par)
                    if b < n_batch - 1:
                        fire_gather(g, b + 1, 1 - par)
                    elif g_par == 0:
                        fire_gather(g + 1, 0, 1 - par)
                    else:
                        @pl.when(gg < n_chunks // 2 - 1)
                        def _():
                            fire_gather(g + 1, 0, 1 - par)
                    add_pos(par, pg)
                    fire_out(g, b, par)
            return carry

        lax.fori_loop(0, n_chunks // 2, gg_body, 0)
        wait_out((n_batch * n_chunks - 1) % 2)

    return k


def kernel(x_ids, token_table, pos_table):
    b, t = x_ids.shape
    _, d = token_table.shape
    flat_ids = x_ids.reshape(b * t).astype(jnp.int32)
    info = plsc.get_sparse_core_info()
    n_workers = info.num_cores * info.num_subcores
    k = _embed_kernel(b, t, d, n_workers, info.num_cores)
    out = k(flat_ids, token_table, pos_table)
    return out.reshape(b, t, d)


# speedup vs baseline: 1.3406x; 1.2081x over previous
import functools

import jax
import jax.numpy as jnp
from jax import lax
from jax.experimental import pallas as pl
from jax.experimental.pallas import tpu as pltpu
from jax.experimental.pallas import tpu_sc as plsc

_LANES = 16
_CHUNK = 16


def _embed_kernel(n_batch, seq_len, d_model, n_workers, n_cores):
    pos_per_w = seq_len // n_workers
    n_chunks = pos_per_w // _CHUNK
    vregs_per_row = d_model // _LANES
    assert n_chunks % 2 == 0 and n_batch % 2 == 0

    mesh = plsc.VectorSubcoreMesh(core_axis_name="c", subcore_axis_name="s")

    @functools.partial(
        pl.kernel,
        mesh=mesh,
        out_type=jax.ShapeDtypeStruct((n_batch * seq_len, d_model), jnp.float32),
        scratch_types=[
            pltpu.VMEM((n_batch, pos_per_w), jnp.int32),
            pltpu.VMEM((2, _CHUNK, d_model), jnp.float32),
            pltpu.VMEM((2, _CHUNK, d_model), jnp.float32),
            pltpu.SemaphoreType.DMA,
            pltpu.SemaphoreType.DMA,
            pltpu.SemaphoreType.DMA,
            pltpu.SemaphoreType.DMA,
            pltpu.SemaphoreType.DMA,
            pltpu.SemaphoreType.DMA,
        ],
    )
    def k(ids_hbm, tok_hbm, pos_hbm, out_hbm, idx_v, tbuf, pbuf,
          g0, g1, o0, o1, p0, p1):
        wid = lax.axis_index("s") * n_cores + lax.axis_index("c")
        pbase = wid * pos_per_w
        gsem, osem, psem = (g0, g1), (o0, o1), (p0, p1)

        for b in range(n_batch):
            pltpu.sync_copy(
                ids_hbm.at[pl.ds(b * seq_len + pbase, pos_per_w)], idx_v.at[b]
            )

        def fire_pos(g, pg):
            pltpu.async_copy(
                pos_hbm.at[pl.ds(pbase + g * _CHUNK, _CHUNK)], pbuf.at[pg],
                psem[pg],
            )

        def wait_pos(pg):
            pltpu.make_async_copy(
                pos_hbm.at[pl.ds(0, _CHUNK)], pbuf.at[pg], psem[pg]
            ).wait()

        def fire_gather(g, b, par):
            pltpu.async_copy(
                tok_hbm.at[idx_v.at[b, pl.ds(g * _CHUNK, _CHUNK)]],
                tbuf.at[par], gsem[par],
            )

        def wait_gather(par):
            pltpu.make_async_copy(
                tok_hbm.at[pl.ds(0, _CHUNK)], tbuf.at[par], gsem[par]
            ).wait()

        def fire_out(g, b, par):
            pltpu.async_copy(
                tbuf.at[par],
                out_hbm.at[pl.ds(b * seq_len + pbase + g * _CHUNK, _CHUNK)],
                osem[par],
            )

        def wait_out(par):
            pltpu.make_async_copy(
                tbuf.at[par], out_hbm.at[pl.ds(0, _CHUNK)], osem[par]
            ).wait()

        def add_pos(par, pg):
            def row_body(r, c):
                for j in range(vregs_per_row):
                    sl = pl.ds(j * _LANES, _LANES)
                    plsc.addupdate(tbuf.at[par, r, sl], pbuf[pg, r, sl])
                return c

            lax.fori_loop(0, _CHUNK, row_body, 0)

        fire_pos(0, 0)
        fire_gather(0, 0, 0)

        def gg_body(gg, carry):
            for g_par in (0, 1):
                g = 2 * gg + g_par
                pg = g_par
                for b in range(n_batch):
                    par = b % 2
                    wait_gather(par)
                    if b == 0:
                        wait_pos(pg)
                        if g_par == 1:
                            @pl.when(gg < n_chunks // 2 - 1)
                            def _():
                                fire_pos(g + 1, 1 - pg)
                        else:
                            fire_pos(g + 1, 1 - pg)
                    if b == 0 and g_par == 0:
                        @pl.when(gg > 0)
                        def _():
                            wait_out(1 - par)
                    else:
                        wait_out(1 - par)
                    if b < n_batch - 1:
                        fire_gather(g, b + 1, 1 - par)
                    elif g_par == 0:
                        fire_gather(g + 1, 0, 1 - par)
                    else:
                        @pl.when(gg < n_chunks // 2 - 1)
                        def _():
                            fire_gather(g + 1, 0, 1 - par)
                    add_pos(par, pg)
                    fire_out(g, b, par)
            return carry

        lax.fori_loop(0, n_chunks // 2, gg_body, 0)
        wait_out((n_batch * n_chunks - 1) % 2)

    return k


def kernel(x_ids, token_table, pos_table):
    b, t = x_ids.shape
    _, d = token_table.shape
    flat_ids = x_ids.reshape(b * t).astype(jnp.int32)
    info = plsc.get_sparse_core_info()
    n_workers = info.num_cores * info.num_subcores
    k = _embed_kernel(b, t, d, n_workers, info.num_cores)
    out = k(flat_ids, token_table, pos_table)
    return out.reshape(b, t, d)
